# Initial kernel scaffold; baseline (speedup 1.0000x reference)
import jax
import jax.numpy as jnp
from jax.experimental import pallas as pl


def kernel(x, W1, g1, b1, W2, g2, b2, W3, g3, b3, W4, g4, b4, W5, g5, b5, L1, g6, b6, L2, L2b, g7, b7, L3, L3b):
    raise NotImplementedError("write your pallas kernel here")


# speedup vs baseline: 1.2928x; 1.2928x over previous
import functools

import jax
import jax.numpy as jnp
import numpy as np
from jax import lax
from jax.experimental import pallas as pl

K = 20
_SC = float(np.float32(1.0) / np.sqrt(np.float32(1.0 + 1e-5)))
_NEG = -1e30
_HI = lax.Precision.HIGHEST


def _bf(x):
    return x.astype(jnp.bfloat16)


def _bfr(x):
    return x.astype(jnp.bfloat16).astype(jnp.float32)


def _lrelu(v):
    return jnp.where(v >= 0, v, 0.2 * v)


def _jacobi3(a11, a12, a13, a22, a23, a33):
    one = jnp.ones_like(a11)
    zero = jnp.zeros_like(a11)
    v = [[one, zero, zero], [zero, one, zero], [zero, zero, one]]
    A = {(0, 0): a11, (0, 1): a12, (0, 2): a13,
         (1, 1): a22, (1, 2): a23, (2, 2): a33}

    def getA(i, j):
        return A[(min(i, j), max(i, j))]

    for _ in range(6):
        for (p, q) in ((0, 1), (0, 2), (1, 2)):
            apq = getA(p, q)
            app = getA(p, p)
            aqq = getA(q, q)
            nz = apq != 0.0
            apq_s = jnp.where(nz, apq, one)
            tau = (aqq - app) / (2.0 * apq_s)
            sgn = jnp.where(tau >= 0, one, -one)
            t = sgn / (jnp.abs(tau) + jnp.sqrt(1.0 + tau * tau))
            t = jnp.where(nz, t, zero)
            c = 1.0 / jnp.sqrt(1.0 + t * t)
            s = t * c
            A[(p, p)] = app - t * apq
            A[(q, q)] = aqq + t * apq
            A[(p, q)] = zero
            r = 3 - p - q
            arp = getA(r, p)
            arq = getA(r, q)
            A[(min(r, p), max(r, p))] = c * arp - s * arq
            A[(min(r, q), max(r, q))] = s * arp + c * arq
            for i in range(3):
                vip = v[i][p]
                viq = v[i][q]
                v[i][p] = c * vip - s * viq
                v[i][q] = s * vip + c * viq
    return (getA(0, 0), getA(1, 1), getA(2, 2)), v


def _frame_fix(nv):
    one = jnp.ones_like(nv[0][0])
    for r in range(3):
        u = [nv[0][r], nv[1][r], nv[2][r]]
        au = [jnp.abs(u[0]), jnp.abs(u[1]), jnp.abs(u[2])]
        m = jnp.maximum(jnp.maximum(au[0], au[1]), au[2])
        sel0 = au[0] == m
        sel1 = (~sel0) & (au[1] == m)
        val = jnp.where(sel0, u[0], jnp.where(sel1, u[1], u[2]))
        s = jnp.where(val >= 0, one, -one)
        for i in range(3):
            nv[i][r] = nv[i][r] * s
    det = (nv[0][0] * (nv[1][1] * nv[2][2] - nv[1][2] * nv[2][1])
           - nv[0][1] * (nv[1][0] * nv[2][2] - nv[1][2] * nv[2][0])
           + nv[0][2] * (nv[1][0] * nv[2][1] - nv[1][1] * nv[2][0]))
    flip = jnp.where(det < 0, -one, one)
    for i in range(3):
        nv[i][2] = nv[i][2] * flip
    return nv


def _skew_signs(sk):
    one = jnp.ones_like(sk[0])
    s = [jnp.where(sk[j] > 0, one, jnp.where(sk[j] < 0, -one, one))
         for j in range(3)]
    neg = sum(jnp.where(sj < 0, one, 0.0) for sj in s)
    odd = (neg == 1.0) | (neg == 3.0)
    s[2] = jnp.where(odd, -s[2], s[2])
    return s


def _canon_coords(pc):
    M, Nk, _ = pc.shape
    centered = pc - jnp.mean(pc, axis=1, keepdims=True)
    cov = jnp.einsum('mki,mkj->mij', centered, centered) / (Nk - 1)
    _, vecs = jnp.linalg.eigh(cov)
    vecs = vecs[:, :, ::-1]
    max_idx = jnp.argmax(jnp.abs(vecs), axis=1, keepdims=True)
    max_vals = jnp.take_along_axis(vecs, max_idx, axis=1)
    signs = jnp.sign(max_vals)
    signs = jnp.where(signs == 0, jnp.ones_like(signs), signs)
    vecs = vecs * signs
    det = jnp.linalg.det(vecs)
    flip = (det < 0).astype(vecs.dtype)
    col = 1.0 - 2.0 * flip
    scale = jnp.stack([jnp.ones_like(col), jnp.ones_like(col), col], axis=-1)
    vecs = vecs * scale[:, None, :]
    cp = jnp.einsum('mki,mij->mkj', centered, vecs)
    skew = jnp.mean(cp ** 3, axis=1)
    s = jnp.sign(skew)
    s = jnp.where(s == 0, jnp.ones_like(s), s)
    odd = (jnp.sum((s < 0).astype(jnp.int32), axis=-1) % 2) == 1
    fix = jnp.stack([jnp.ones(odd.shape, s.dtype),
                     jnp.ones(odd.shape, s.dtype),
                     jnp.where(odd, -1.0, 1.0).astype(s.dtype)], axis=-1)
    s = s * fix
    return cp * s[:, None, :]


def _global_body(x_ref, vec_ref, o_ref, *, n):
    xt = x_ref[0]
    v9 = vec_ref[0]
    mean = jnp.sum(xt, axis=1, keepdims=True) / float(n)
    c = xt - mean
    nv = [[v9[0:1, i * 3 + (2 - j):i * 3 + (2 - j) + 1] for j in range(3)]
          for i in range(3)]
    nv = _frame_fix(nv)
    cb = [_bfr(c[i:i + 1, :]) for i in range(3)]
    nvb = [[_bfr(nv[i][j]) for j in range(3)] for i in range(3)]
    cp = [nvb[0][j] * cb[0] + nvb[1][j] * cb[1] + nvb[2][j] * cb[2]
          for j in range(3)]
    sk = [jnp.sum((cp[j] * cp[j]) * cp[j], axis=1, keepdims=True) / float(n)
          for j in range(3)]
    s = _skew_signs(sk)
    cp2 = [cp[j] * s[j] for j in range(3)]
    kx, ky, kz = cp2
    kxc = jnp.transpose(kx)
    kyc = jnp.transpose(ky)
    kzc = jnp.transpose(kz)
    jj = lax.broadcasted_iota(jnp.int32, (n, n), 0)
    ii = lax.broadcasted_iota(jnp.int32, (n, n), 1)
    less = ((kxc < kx)
            | ((kxc == kx)
               & ((kyc < ky)
                  | ((kyc == ky)
                     & ((kzc < kz) | ((kzc == kz) & (jj < ii)))))))
    rank = jnp.sum(less.astype(jnp.float32), axis=0, keepdims=True)
    rr = lax.broadcasted_iota(jnp.int32, (1, n), 1).astype(jnp.float32)
    P = (jnp.transpose(rank) == rr).astype(jnp.float32)
    cp2m = jnp.concatenate(cp2, axis=0)
    ordered = lax.dot_general(P, cp2m, (((0,), (1,)), ((), ())),
                              precision=_HI,
                              preferred_element_type=jnp.float32)
    o_ref[0] = ordered


def _global_canon(x):
    B, _, N = x.shape
    pc = jnp.transpose(x, (0, 2, 1))
    centered = pc - jnp.mean(pc, axis=1, keepdims=True)
    cov = jnp.einsum('mki,mkj->mij', centered, centered) / (N - 1)
    _, vecs = jnp.linalg.eigh(cov)
    vec9 = jnp.reshape(vecs, (B, 1, 9))
    return pl.pallas_call(
        functools.partial(_global_body, n=N),
        grid=(B,),
        in_specs=[pl.BlockSpec((1, 3, N), lambda b: (b, 0, 0)),
                  pl.BlockSpec((1, 1, 9), lambda b: (b, 0, 0))],
        out_specs=pl.BlockSpec((1, N, 3), lambda b: (b, 0, 0)),
        out_shape=jax.ShapeDtypeStruct((B, N, 3), jnp.float32),
    )(x, vec9)


def _knn_body(fa_ref, fr_ref, pts_ref, co_ref, ix_ref, *, n, rb):
    fa = fa_ref[0]
    fr = fr_ref[0]
    pts = pts_ref[0]
    gmat = lax.dot_general(_bf(fr), _bf(fa), (((1,), (1,)), ((), ())),
                           preferred_element_type=jnp.float32)
    inner = -2.0 * gmat
    xxr = jnp.sum(fr * fr, axis=1, keepdims=True)
    xxa = jnp.transpose(jnp.sum(fa * fa, axis=1, keepdims=True))
    pd = -xxr - inner - xxa
    iota = lax.broadcasted_iota(jnp.int32, (rb, n), 1)
    coords = []
    idxs = []
    for _ in range(K):
        m = jnp.max(pd, axis=1, keepdims=True)
        cand = jnp.where(pd >= m, iota, n)
        am = jnp.min(cand, axis=1, keepdims=True)
        oh = (iota == am)
        ohf = oh.astype(jnp.float32)
        coords.append(lax.dot_general(ohf, pts, (((1,), (0,)), ((), ())),
                                      precision=_HI,
                                      preferred_element_type=jnp.float32))
        idxs.append(am)
        pd = jnp.where(oh, _NEG, pd)
    co_ref[0] = jnp.concatenate(coords, axis=1)
    ix_ref[0] = jnp.concatenate(idxs, axis=1)


def _knn_gather(feats, pts, rb=256):
    B, N, C = feats.shape
    body = functools.partial(_knn_body, n=N, rb=rb)
    return pl.pallas_call(
        body,
        grid=(B, N // rb),
        in_specs=[
            pl.BlockSpec((1, N, C), lambda bi, ni: (bi, 0, 0)),
            pl.BlockSpec((1, rb, C), lambda bi, ni: (bi, ni, 0)),
            pl.BlockSpec((1, N, 3), lambda bi, ni: (bi, 0, 0)),
        ],
        out_specs=[
            pl.BlockSpec((1, rb, K * 3), lambda bi, ni: (bi, ni, 0)),
            pl.BlockSpec((1, rb, K), lambda bi, ni: (bi, ni, 0)),
        ],
        out_shape=[
            jax.ShapeDtypeStruct((B, N, K * 3), jnp.float32),
            jax.ShapeDtypeStruct((B, N, K), jnp.int32),
        ],
    )(feats, feats, pts)


def _layer_body(cp_ref, ix_ref, fa_ref, w_ref, g_ref, bb_ref,
                o_ref, *, n, rb, c_extra, out_ch):
    cc = 3 + c_extra
    fa = fa_ref[0]
    cpt = cp_ref[0]
    IDX = ix_ref[0]

    kx = jnp.concatenate([cpt[3 * k + 0:3 * k + 1, :] for k in range(K)],
                         axis=0)
    ky = jnp.concatenate([cpt[3 * k + 1:3 * k + 2, :] for k in range(K)],
                         axis=0)
    kz = jnp.concatenate([cpt[3 * k + 2:3 * k + 3, :] for k in range(K)],
                         axis=0)

    iota_k = lax.broadcasted_iota(jnp.int32, (K, 1), 0)
    rank = jnp.zeros((K, rb), jnp.float32)
    for j in range(K):
        xj = kx[j:j + 1, :]
        yj = ky[j:j + 1, :]
        zj = kz[j:j + 1, :]
        before = ((xj < kx)
                  | ((xj == kx)
                     & ((yj < ky)
                        | ((yj == ky)
                           & ((zj < kz) | ((zj == kz) & (j < iota_k)))))))
        rank = rank + before.astype(jnp.float32)

    iota_nr = lax.broadcasted_iota(jnp.int32, (n, rb), 0)
    acc = jnp.zeros((rb, out_ch), jnp.float32)
    for r in range(K):
        rf = jnp.float32(r)
        selx = jnp.zeros((1, rb), jnp.float32)
        sely = jnp.zeros((1, rb), jnp.float32)
        selz = jnp.zeros((1, rb), jnp.float32)
        seli = jnp.zeros((1, rb), jnp.int32)
        for k in range(K):
            mk = rank[k:k + 1, :] == rf
            selx = selx + jnp.where(mk, kx[k:k + 1, :], 0.0)
            sely = sely + jnp.where(mk, ky[k:k + 1, :], 0.0)
            selz = selz + jnp.where(mk, kz[k:k + 1, :], 0.0)
            seli = seli + jnp.where(mk, IDX[k:k + 1, :], 0)
        if c_extra == 0:
            sel_t = jnp.concatenate([selx, sely, selz], axis=0)
        else:
            ohs = (iota_nr == seli).astype(jnp.float32)
            vf = lax.dot_general(fa, ohs, (((0,), (0,)), ((), ())),
                                 precision=_HI,
                                 preferred_element_type=jnp.float32)
            sel_t = jnp.concatenate([selx, sely, selz, vf], axis=0)
        acc = acc + lax.dot_general(
            _bf(sel_t), _bf(w_ref[:, r * cc:(r + 1) * cc]),
            (((0,), (1,)), ((), ())),
            preferred_element_type=jnp.float32)
    out = acc * _SC * g_ref[0:1, :] + bb_ref[0:1, :]
    o_ref[0] = _lrelu(out)


def _layer(feats, pts, W, g, b, c_extra, rb=256):
    B, N, C = feats.shape
    O = W.shape[0]
    cc = 3 + c_extra
    g2 = jnp.reshape(g, (1, O))
    b2 = jnp.reshape(b, (1, O))
    co, ix = _knn_gather(feats, pts, rb=rb)
    patch = jnp.reshape(co, (B * N, K, 3))
    cp2 = _canon_coords(patch)
    cp_t = jnp.transpose(jnp.reshape(cp2, (B, N, K * 3)), (0, 2, 1))
    ix_t = jnp.transpose(ix, (0, 2, 1))
    body = functools.partial(_layer_body, n=N, rb=rb, c_extra=c_extra,
                             out_ch=O)
    return pl.pallas_call(
        body,
        grid=(B, N // rb),
        in_specs=[
            pl.BlockSpec((1, K * 3, rb), lambda bi, ni: (bi, 0, ni)),
            pl.BlockSpec((1, K, rb), lambda bi, ni: (bi, 0, ni)),
            pl.BlockSpec((1, N, C), lambda bi, ni: (bi, 0, 0)),
            pl.BlockSpec((O, K * cc), lambda bi, ni: (0, 0)),
            pl.BlockSpec((1, O), lambda bi, ni: (0, 0)),
            pl.BlockSpec((1, O), lambda bi, ni: (0, 0)),
        ],
        out_specs=pl.BlockSpec((1, rb, O), lambda bi, ni: (bi, ni, 0)),
        out_shape=jax.ShapeDtypeStruct((B, N, O), jnp.float32),
    )(cp_t, ix_t, feats, W, g2, b2)


def _pool_body(x1_ref, x2_ref, x3_ref, x4_ref, w_ref, g_ref, bb_ref, o_ref,
               *, n):
    xc = jnp.concatenate([x1_ref[0], x2_ref[0], x3_ref[0], x4_ref[0]],
                         axis=1)
    z = jnp.dot(_bf(xc), _bf(w_ref[...]), preferred_element_type=jnp.float32)
    z = _lrelu(z * _SC * g_ref[0:1, :] + bb_ref[0:1, :])
    mx = jnp.max(z, axis=0, keepdims=True)
    mn = jnp.sum(z, axis=0, keepdims=True) / float(n)
    o_ref[0] = jnp.concatenate([mx, mn], axis=1)


def _pool(x1, x2, x3, x4, W5, g5, b5):
    B, N, _ = x1.shape
    emb = W5.shape[0]
    W5t = jnp.transpose(W5)
    g2 = jnp.reshape(g5, (1, emb))
    b2 = jnp.reshape(b5, (1, emb))
    return pl.pallas_call(
        functools.partial(_pool_body, n=N),
        grid=(B,),
        in_specs=[
            pl.BlockSpec((1, N, x1.shape[2]), lambda b_: (b_, 0, 0)),
            pl.BlockSpec((1, N, x2.shape[2]), lambda b_: (b_, 0, 0)),
            pl.BlockSpec((1, N, x3.shape[2]), lambda b_: (b_, 0, 0)),
            pl.BlockSpec((1, N, x4.shape[2]), lambda b_: (b_, 0, 0)),
            pl.BlockSpec((W5t.shape[0], emb), lambda b_: (0, 0)),
            pl.BlockSpec((1, emb), lambda b_: (0, 0)),
            pl.BlockSpec((1, emb), lambda b_: (0, 0)),
        ],
        out_specs=pl.BlockSpec((1, 1, 2 * emb), lambda b_: (b_, 0, 0)),
        out_shape=jax.ShapeDtypeStruct((B, 1, 2 * emb), jnp.float32),
    )(x1, x2, x3, x4, W5t, g2, b2)[:, 0, :]


def _mlp_body(xp_ref, l1_ref, g6_ref, b6_ref, l2_ref, l2b_ref, g7_ref,
              b7_ref, l3_ref, l3b_ref, o_ref):
    h = jnp.dot(_bf(xp_ref[...]), _bf(l1_ref[...]),
                preferred_element_type=jnp.float32)
    h = _lrelu(h * _SC * g6_ref[0:1, :] + b6_ref[0:1, :])
    h = jnp.dot(_bf(h), _bf(l2_ref[...]), preferred_element_type=jnp.float32)
    h = h + l2b_ref[0:1, :]
    h = _lrelu(h * _SC * g7_ref[0:1, :] + b7_ref[0:1, :])
    out = jnp.dot(_bf(h), _bf(l3_ref[...]),
                  preferred_element_type=jnp.float32)
    o_ref[...] = out + l3b_ref[0:1, :]


def _mlp(xp, L1, g6, b6, L2, L2b, g7, b7, L3, L3b):
    B = xp.shape[0]
    r1 = lambda a: jnp.reshape(a, (1, a.shape[0]))
    args = (xp, jnp.transpose(L1), r1(g6), r1(b6), jnp.transpose(L2),
            r1(L2b), r1(g7), r1(b7), jnp.transpose(L3), r1(L3b))
    return pl.pallas_call(
        _mlp_body,
        in_specs=[pl.BlockSpec(a.shape, lambda: tuple(0 for _ in a.shape))
                  for a in args],
        out_specs=pl.BlockSpec((B, L3.shape[0]), lambda: (0, 0)),
        out_shape=jax.ShapeDtypeStruct((B, L3.shape[0]), jnp.float32),
    )(*args)


def kernel(x, W1, g1, b1, W2, g2, b2, W3, g3, b3, W4, g4, b4, W5, g5, b5,
           L1, g6, b6, L2, L2b, g7, b7, L3, L3b):
    pts = _global_canon(x)
    x1 = _layer(pts, pts, W1, g1, b1, c_extra=0)
    x2 = _layer(x1, pts, W2, g2, b2, c_extra=64)
    x3 = _layer(x2, pts, W3, g3, b3, c_extra=64)
    x4 = _layer(x3, pts, W4, g4, b4, c_extra=128)
    xp = _pool(x1, x2, x3, x4, W5, g5, b5)
    return _mlp(xp, L1, g6, b6, L2, L2b, g7, b7, L3, L3b)
